# baseline (speedup 1.0000x reference)
import functools

import jax
import jax.numpy as jnp
from jax import lax
from jax.experimental import pallas as pl
from jax.experimental.pallas import tpu as pltpu
from jax.experimental.pallas import tpu_sc as plsc

_B = 16384
_HIST = 50
_D = 64
_NC = 2
_NS = 16
_NW = _NC * _NS
_BAGS_PER_W = _B // _NW
_CPB = 2
_IPC = _CPB * _HIST
_CHUNKS = _BAGS_PER_W // _CPB
_NREG = _D // 16


def _sc_embedding_sum(x2d, table, emb_bias):
    mesh = plsc.VectorSubcoreMesh(
        core_axis_name="c", subcore_axis_name="s",
        num_cores=_NC, num_subcores=_NS,
    )

    @functools.partial(
        pl.kernel,
        out_type=jax.ShapeDtypeStruct((_B, _D), jnp.float32),
        mesh=mesh,
        compiler_params=pltpu.CompilerParams(use_tc_tiling_on_sc=False),
        scratch_types=[
            pltpu.VMEM((_CHUNKS, _IPC), jnp.int32),
            pltpu.VMEM((_IPC, _D), jnp.float32),
            pltpu.VMEM((_IPC, _D), jnp.float32),
            pltpu.VMEM((_IPC, _D), jnp.float32),
            pltpu.VMEM((_IPC, _D), jnp.float32),
            pltpu.VMEM((_BAGS_PER_W, _D), jnp.float32),
            pltpu.VMEM((_D,), jnp.float32),
            pltpu.SemaphoreType.DMA,
            pltpu.SemaphoreType.DMA,
            pltpu.SemaphoreType.DMA,
            pltpu.SemaphoreType.DMA,
        ],
    )
    def k(x_hbm, tab_hbm, bias_hbm, out_hbm,
          idx_v, rows0, rows1, rows2, rows3, out_v, bias_v,
          sem0, sem1, sem2, sem3):
        wid = lax.axis_index("s") * _NC + lax.axis_index("c")
        pltpu.sync_copy(x_hbm.at[pl.ds(wid * _CHUNKS, _CHUNKS)], idx_v)
        pltpu.sync_copy(bias_hbm, bias_v)
        bias_regs = [bias_v[pl.ds(16 * g, 16)] for g in range(_NREG)]

        def start(j, rows, sem):
            pltpu.async_copy(tab_hbm.at[idx_v.at[j]], rows, sem)

        def wait(j, rows, sem):
            pltpu.make_async_copy(tab_hbm.at[idx_v.at[j]], rows, sem).wait()

        def reduce_chunk(j, rows):
            for bag in range(_CPB):
                acc_a = list(bias_regs)
                acc_b = [rows[bag * _HIST + 1, pl.ds(16 * g, 16)]
                         for g in range(_NREG)]
                for l in range(_HIST):
                    if l == 1:
                        continue
                    r = bag * _HIST + l
                    if l % 2 == 0:
                        acc_a = [acc_a[g] + rows[r, pl.ds(16 * g, 16)]
                                 for g in range(_NREG)]
                    else:
                        acc_b = [acc_b[g] + rows[r, pl.ds(16 * g, 16)]
                                 for g in range(_NREG)]
                ob = j * _CPB + bag
                for g in range(_NREG):
                    out_v[ob, pl.ds(16 * g, 16)] = acc_a[g] + acc_b[g]

        bufs = (rows0, rows1, rows2, rows3)
        sems = (sem0, sem1, sem2, sem3)
        _DEPTH = 4
        for kk in range(_DEPTH - 1):
            start(kk, bufs[kk], sems[kk])

        def step(i, carry):
            base = _DEPTH * i
            for kk in range(_DEPTH):
                j = base + kk
                nxt = j + (_DEPTH - 1)

                @pl.when(nxt < _CHUNKS)
                def _prefetch():
                    start(nxt, bufs[(kk + _DEPTH - 1) % _DEPTH],
                          sems[(kk + _DEPTH - 1) % _DEPTH])

                wait(j, bufs[kk], sems[kk])
                reduce_chunk(j, bufs[kk])
            return carry

        lax.fori_loop(0, _CHUNKS // _DEPTH, step, 0)
        pltpu.sync_copy(out_v, out_hbm.at[pl.ds(wid * _BAGS_PER_W, _BAGS_PER_W)])

    return k(x2d, table, emb_bias)


def kernel(x, table, emb_bias):
    x2d = x.astype(jnp.int32).reshape(_B * _HIST // _IPC, _IPC)
    return _sc_embedding_sum(x2d, table, emb_bias)
